# baseline (speedup 1.0000x reference)
import jax
import jax.numpy as jnp
from jax.experimental import pallas as pl
from jax.experimental.pallas import tpu as pltpu

_CAP = 100000
_DIM = 128
_ROWS = 4000


def _store_body(wp_ref, state_ref, mem_ref, out_ref, mean_ref):
    i = pl.program_id(0)

    @pl.when(i == 0)
    def _():
        mean_ref[...] = jnp.mean(state_ref[...], axis=0, keepdims=True)

    out_ref[...] = mem_ref[...]

    idx = wp_ref[0]
    lo = i * _ROWS

    @pl.when((idx >= lo) & (idx < lo + _ROWS))
    def _():
        out_ref[pl.ds(idx - lo, 1), :] = mean_ref[...]


def kernel(state, memory_states, write_pointer, is_full):
    new_memory = pl.pallas_call(
        _store_body,
        grid_spec=pltpu.PrefetchScalarGridSpec(
            num_scalar_prefetch=1,
            grid=(_CAP // _ROWS,),
            in_specs=[
                pl.BlockSpec((4096, _DIM), lambda i, wp: (0, 0)),
                pl.BlockSpec((_ROWS, _DIM), lambda i, wp: (i, 0)),
            ],
            out_specs=pl.BlockSpec((_ROWS, _DIM), lambda i, wp: (i, 0)),
            scratch_shapes=[pltpu.VMEM((1, _DIM), jnp.float32)],
        ),
        out_shape=jax.ShapeDtypeStruct((_CAP, _DIM), jnp.float32),
    )(write_pointer, state, memory_states)

    nxt = write_pointer[0] + 1
    new_pointer = write_pointer.at[0].set(nxt % _CAP)
    new_is_full = jnp.where(nxt == _CAP, jnp.ones_like(is_full), is_full)
    return new_memory, new_pointer, new_is_full
